# baseline (speedup 1.0000x reference)
import functools

import jax
import jax.numpy as jnp
from jax import lax
from jax.experimental import pallas as pl
from jax.experimental.pallas import tpu as pltpu
from jax.experimental.pallas import tpu_sc as plsc

BATCH = 16384
K = 16
LANE = 128
NUM_CORES = 2
NUM_SUBCORES = 16
NW = NUM_CORES * NUM_SUBCORES
BPW = BATCH // NW
NGRP = BPW // K


def _mf_body(user_hbm, item_hbm, wt_hbm, ht_hbm, out_hbm,
             uidx_v, iidx_v, win_v, urt_v, vrt_v, out_v, sem):
    c = lax.axis_index("c")
    s = lax.axis_index("s")
    wid = s * NUM_CORES + c

    pltpu.sync_copy(user_hbm.at[wid], uidx_v)
    pltpu.sync_copy(item_hbm.at[wid], iidx_v)

    iota = lax.iota(jnp.int32, K)

    def gather_pass(idx_v, tab_hbm, rt_v):
        def fire(g, b):
            vv = idx_v[pl.ds(g * K, K)] & jnp.int32(~(LANE - 1))
            for r in range(K):
                off = pl.multiple_of(vv[r], LANE)
                pltpu.async_copy(tab_hbm.at[:, pl.ds(off, LANE)],
                                 win_v.at[b, r], sem)

        fire(0, 0)
        fire(1, 1)

        def group_step(g, _):
            b = lax.rem(g, 3)

            @pl.when(g + 2 < NGRP)
            def _fire_next():
                fire(g + 2, lax.rem(g + 2, 3))

            for r in range(K):
                pltpu.make_async_copy(tab_hbm.at[:, pl.ds(0, LANE)],
                                      win_v.at[b, r], sem).wait()

            rem = idx_v[pl.ds(g * K, K)] & jnp.int32(LANE - 1)
            bvec = jnp.full((K,), b, jnp.int32)
            for j in range(K):
                cols = jnp.full((K,), j, jnp.int32)
                vals = plsc.load_gather(win_v, [bvec, iota, cols, rem])
                rt_v[j, pl.ds(g * K, K)] = vals
            return _

        lax.fori_loop(0, NGRP, group_step, 0)

    gather_pass(uidx_v, wt_hbm, urt_v)
    gather_pass(iidx_v, ht_hbm, vrt_v)

    def dot_step(g, _):
        acc = jnp.zeros((K,), jnp.float32)
        for j in range(K):
            acc = acc + urt_v[j, pl.ds(g * K, K)] * vrt_v[j, pl.ds(g * K, K)]
        out_v[pl.ds(g * K, K)] = acc
        return _

    lax.fori_loop(0, NGRP, dot_step, 0)
    pltpu.sync_copy(out_v, out_hbm.at[wid])


@functools.partial(jax.jit, static_argnums=())
def _mf(user, item, wt, ht):
    mesh = plsc.VectorSubcoreMesh(core_axis_name="c", subcore_axis_name="s")
    f = pl.kernel(
        _mf_body,
        out_type=jax.ShapeDtypeStruct((NW, BPW), jnp.float32),
        mesh=mesh,
        scratch_types=[
            pltpu.VMEM((BPW,), jnp.int32),
            pltpu.VMEM((BPW,), jnp.int32),
            pltpu.VMEM((3, K, K, LANE), jnp.float32),
            pltpu.VMEM((K, BPW), jnp.float32),
            pltpu.VMEM((K, BPW), jnp.float32),
            pltpu.VMEM((BPW,), jnp.float32),
            pltpu.SemaphoreType.DMA,
        ],
        compiler_params=pltpu.CompilerParams(
            needs_layout_passes=False, use_tc_tiling_on_sc=True),
    )
    return f(user, item, wt, ht)


def kernel(x, W, H):
    xi = x.astype(jnp.int32)
    user = xi[:, 0].reshape(NW, BPW)
    item = xi[:, 1].reshape(NW, BPW)
    out = _mf(user, item, W.T, H.T)
    return out.reshape(BATCH)


# speedup vs baseline: 1.0223x; 1.0223x over previous
import functools

import jax
import jax.numpy as jnp
from jax import lax
from jax.experimental import pallas as pl
from jax.experimental.pallas import tpu as pltpu
from jax.experimental.pallas import tpu_sc as plsc

BATCH = 16384
K = 16
LANE = 128
NUM_CORES = 2
NUM_SUBCORES = 16
NW = NUM_CORES * NUM_SUBCORES
BPW = BATCH // NW
NGRP = BPW // K


def _mf_body(user_hbm, item_hbm, wt_hbm, ht_hbm, out_hbm,
             uidx_v, iidx_v, win_v, urt_v, vrt_v, out_v, sem):
    c = lax.axis_index("c")
    s = lax.axis_index("s")
    wid = s * NUM_CORES + c

    pltpu.sync_copy(user_hbm.at[wid], uidx_v)
    pltpu.sync_copy(item_hbm.at[wid], iidx_v)

    iota = lax.iota(jnp.int32, K)

    def gather_pass(idx_v, tab_hbm, rt_v):
        def fire(g, b):
            vv = idx_v[pl.ds(g * K, K)] & jnp.int32(~(LANE - 1))
            for r in range(K):
                off = pl.multiple_of(vv[r], LANE)
                pltpu.async_copy(tab_hbm.at[:, pl.ds(off, LANE)],
                                 win_v.at[b, r], sem)

        fire(0, 0)

        def group_step(g, _):
            b = lax.rem(g, 2)

            @pl.when(g + 1 < NGRP)
            def _fire_next():
                fire(g + 1, lax.rem(g + 1, 2))

            for r in range(K):
                pltpu.make_async_copy(tab_hbm.at[:, pl.ds(0, LANE)],
                                      win_v.at[b, r], sem).wait()

            rem = idx_v[pl.ds(g * K, K)] & jnp.int32(LANE - 1)
            bvec = jnp.full((K,), b, jnp.int32)
            for j in range(K):
                cols = jnp.full((K,), j, jnp.int32)
                vals = plsc.load_gather(win_v, [bvec, iota, cols, rem])
                rt_v[j, pl.ds(g * K, K)] = vals
            return _

        lax.fori_loop(0, NGRP, group_step, 0)

    gather_pass(uidx_v, wt_hbm, urt_v)
    gather_pass(iidx_v, ht_hbm, vrt_v)

    def dot_step(g, _):
        acc = jnp.zeros((K,), jnp.float32)
        for j in range(K):
            acc = acc + urt_v[j, pl.ds(g * K, K)] * vrt_v[j, pl.ds(g * K, K)]
        out_v[pl.ds(g * K, K)] = acc
        return _

    lax.fori_loop(0, NGRP, dot_step, 0)
    pltpu.sync_copy(out_v, out_hbm.at[wid])


@functools.partial(jax.jit, static_argnums=())
def _mf(user, item, wt, ht):
    mesh = plsc.VectorSubcoreMesh(core_axis_name="c", subcore_axis_name="s")
    f = pl.kernel(
        _mf_body,
        out_type=jax.ShapeDtypeStruct((NW, BPW), jnp.float32),
        mesh=mesh,
        scratch_types=[
            pltpu.VMEM((BPW,), jnp.int32),
            pltpu.VMEM((BPW,), jnp.int32),
            pltpu.VMEM((2, K, K, LANE), jnp.float32),
            pltpu.VMEM((K, BPW), jnp.float32),
            pltpu.VMEM((K, BPW), jnp.float32),
            pltpu.VMEM((BPW,), jnp.float32),
            pltpu.SemaphoreType.DMA,
        ],
        compiler_params=pltpu.CompilerParams(
            needs_layout_passes=False, use_tc_tiling_on_sc=True),
    )
    return f(user, item, wt, ht)


def kernel(x, W, H):
    xi = x.astype(jnp.int32)
    user = xi[:, 0].reshape(NW, BPW)
    item = xi[:, 1].reshape(NW, BPW)
    out = _mf(user, item, W.T, H.T)
    return out.reshape(BATCH)
